# baseline (speedup 1.0000x reference)
import functools
import math

import jax
import jax.numpy as jnp
from jax import lax
from jax.experimental import pallas as pl
from jax.experimental.pallas import tpu as pltpu
from jax.experimental.pallas import tpu_sc as plsc

VOCAB = 100000
D_MODEL = 128
BATCH = 4096
HIST = 50
HIST_PAD = 56
PAIR = 2 * HIST_PAD

NC = 2
NS = 16
NW = NC * NS
B_PER_W = BATCH // NW
NPAIR = B_PER_W // 2
GRP = 4
NGRP = B_PER_W // GRP
NRB = 4
NCB = 2
SCALE = math.sqrt(D_MODEL)

_mesh = plsc.VectorSubcoreMesh(core_axis_name="c", subcore_axis_name="s")


@functools.partial(
    pl.kernel,
    mesh=_mesh,
    out_type=jax.ShapeDtypeStruct((BATCH * HIST, D_MODEL), jnp.float32),
    scratch_types=[
        pltpu.VMEM((NPAIR, PAIR), jnp.int32),
        pltpu.VMEM((NRB, PAIR, D_MODEL), jnp.float32),
        pltpu.VMEM((NCB, GRP * HIST, D_MODEL), jnp.float32),
        pltpu.SemaphoreType.DMA,
        pltpu.SemaphoreType.DMA,
    ],
)
def _emb_sc(x_hbm, w_hbm, out_hbm, idx_v, rows_v, comp_v, gsem, osem):
    wid = lax.axis_index("s") * NC + lax.axis_index("c")
    b0 = wid * B_PER_W

    pltpu.sync_copy(x_hbm.at[pl.ds(wid * NPAIR, NPAIR)], idx_v)

    def gather_start(p, rb):
        pltpu.async_copy(w_hbm.at[idx_v.at[p]], rows_v.at[rb], gsem)

    def gather_wait(p, rb):
        pltpu.make_async_copy(w_hbm.at[idx_v.at[p]], rows_v.at[rb], gsem).wait()

    def out_start(t, cb):
        pltpu.async_copy(
            comp_v.at[cb], out_hbm.at[pl.ds((b0 + GRP * t) * HIST, GRP * HIST)], osem
        )

    def out_wait(t, cb):
        pltpu.make_async_copy(
            comp_v.at[cb],
            out_hbm.at[pl.ds((b0 + GRP * t) * HIST, GRP * HIST)],
            osem,
        ).wait()

    def scale_group(cb, rb_even, rb_odd):
        comp = comp_v.at[cb]
        for g in range(GRP):
            rows = rows_v.at[rb_even if g < 2 else rb_odd]
            base = HIST_PAD * (g % 2)

            @plsc.parallel_loop(0, HIST, unroll=4)
            def _(k):
                for i in range(D_MODEL // 16):
                    sl = pl.ds(16 * i, 16)
                    comp[g * HIST + k, sl] = rows[base + k, sl] * SCALE

    for rb in range(NRB):
        gather_start(rb, rb)

    def outer(tt, _):
        for u in range(NCB):
            t = tt * NCB + u
            p0 = 2 * t
            rb0 = (2 * u) % NRB
            rb1 = (2 * u + 1) % NRB
            cb = u
            gather_wait(p0, rb0)
            gather_wait(p0 + 1, rb1)

            @pl.when(t >= NCB)
            def _():
                out_wait(t - NCB, cb)

            scale_group(cb, rb0, rb1)
            out_start(t, cb)

            @pl.when(p0 + NRB + 1 < NPAIR + 1)
            def _():
                gather_start(p0 + NRB, rb0)
                gather_start(p0 + NRB + 1, rb1)

        return 0

    lax.fori_loop(0, NGRP // NCB, outer, 0)

    for u in range(NCB):
        out_wait(NGRP - NCB + u, u)


def kernel(x, weight):
    xp = jnp.pad(x, ((0, 0), (0, HIST_PAD - HIST))).reshape(BATCH // 2, PAIR)
    return _emb_sc(xp, weight).reshape(BATCH, HIST, D_MODEL)


# speedup vs baseline: 4.6131x; 4.6131x over previous
import functools
import math

import jax
import jax.numpy as jnp
from jax import lax
from jax.experimental import pallas as pl
from jax.experimental.pallas import tpu as pltpu
from jax.experimental.pallas import tpu_sc as plsc

VOCAB = 100000
D_MODEL = 128
BATCH = 4096
HIST = 50
HIST_PAD = 56
PAIR = 2 * HIST_PAD

NC = 2
NS = 16
NW = NC * NS
B_PER_W = BATCH // NW
NPAIR = B_PER_W // 2
GRP = 4
NGRP = B_PER_W // GRP
NRB = 4
NCB = 2
SCALE = math.sqrt(D_MODEL)

_mesh = plsc.VectorSubcoreMesh(core_axis_name="c", subcore_axis_name="s")


@functools.partial(
    pl.kernel,
    mesh=_mesh,
    out_type=jax.ShapeDtypeStruct((BATCH * HIST, D_MODEL), jnp.float32),
    scratch_types=[
        pltpu.VMEM((NPAIR, PAIR), jnp.int32),
        pltpu.VMEM((NRB, PAIR, D_MODEL), jnp.float32),
        pltpu.VMEM((NCB, GRP * HIST, D_MODEL), jnp.float32),
        pltpu.SemaphoreType.DMA,
        pltpu.SemaphoreType.DMA,
    ],
)
def _emb_sc(x_hbm, w_hbm, out_hbm, idx_v, rows_v, comp_v, gsem, osem):
    wid = lax.axis_index("s") * NC + lax.axis_index("c")
    b0 = wid * B_PER_W

    pltpu.sync_copy(x_hbm.at[pl.ds(wid * NPAIR, NPAIR)], idx_v)

    def gather_start(p, rb):
        pltpu.async_copy(w_hbm.at[idx_v.at[p]], rows_v.at[rb], gsem)

    def gather_wait(p, rb):
        pltpu.make_async_copy(w_hbm.at[idx_v.at[p]], rows_v.at[rb], gsem).wait()

    def out_start(t, cb):
        pltpu.async_copy(
            comp_v.at[cb], out_hbm.at[pl.ds((b0 + GRP * t) * HIST, GRP * HIST)], osem
        )

    def out_wait(t, cb):
        pltpu.make_async_copy(
            comp_v.at[cb],
            out_hbm.at[pl.ds((b0 + GRP * t) * HIST, GRP * HIST)],
            osem,
        ).wait()

    def scale_group(cb, rb_even, rb_odd):
        comp = comp_v.at[cb]
        for g in range(GRP):
            rows = rows_v.at[rb_even if g < 2 else rb_odd]
            base = HIST_PAD * (g % 2)

            @plsc.parallel_loop(0, HIST, unroll=4)
            def _(k):
                for i in range(D_MODEL // 16):
                    sl = pl.ds(16 * i, 16)
                    comp[g * HIST + k, sl] = rows[base + k, sl] * SCALE

    for rb in range(NRB):
        gather_start(rb, rb)

    def outer(tt, _):
        for u in range(NCB):
            t = tt * NCB + u
            p0 = 2 * t
            rb0 = (2 * u) % NRB
            rb1 = (2 * u + 1) % NRB
            cb = u
            gather_wait(p0, rb0)
            gather_wait(p0 + 1, rb1)

            @pl.when(t >= NCB)
            def _():
                out_wait(t - NCB, cb)

            scale_group(cb, rb0, rb1)
            out_start(t, cb)

            @pl.when(p0 + NRB + 1 < NPAIR + 1)
            def _():
                gather_start(p0 + NRB, rb0)
                gather_start(p0 + NRB + 1, rb1)

        return 0

    lax.fori_loop(0, NGRP // NCB, outer, 0)

    for u in range(NCB):
        out_wait(NGRP - NCB + u, u)


def kernel(x, weight):
    xp = jnp.concatenate([x, x[:, : HIST_PAD - HIST]], axis=1)
    xp = xp.reshape(BATCH // 2, PAIR)
    return _emb_sc(xp, weight).reshape(BATCH, HIST, D_MODEL)


# speedup vs baseline: 8.0869x; 1.7530x over previous
import functools
import math

import jax
import jax.numpy as jnp
from jax import lax
from jax.experimental import pallas as pl
from jax.experimental.pallas import tpu as pltpu
from jax.experimental.pallas import tpu_sc as plsc

VOCAB = 100000
D_MODEL = 128
BATCH = 4096
HIST = 50
HIST_PAD = 56
PAIR = 2 * HIST_PAD

NC = 2
NS = 16
NW = NC * NS
B_PER_W = BATCH // NW
NPAIR = B_PER_W // 2
GRP = 4
NGRP = B_PER_W // GRP
NRB = 4
NCB = 2
SCALE = math.sqrt(D_MODEL)

_mesh = plsc.VectorSubcoreMesh(core_axis_name="c", subcore_axis_name="s")


@functools.partial(
    pl.kernel,
    mesh=_mesh,
    out_type=jax.ShapeDtypeStruct((BATCH, HIST, D_MODEL), jnp.float32),
    scratch_types=[
        pltpu.VMEM((NPAIR, PAIR), jnp.int32),
        pltpu.VMEM((NRB, PAIR, D_MODEL), jnp.float32),
        pltpu.VMEM((NCB, GRP, HIST, D_MODEL), jnp.float32),
        pltpu.SemaphoreType.DMA,
        pltpu.SemaphoreType.DMA,
    ],
)
def _emb_sc(x_hbm, w_hbm, out_hbm, idx_v, rows_v, comp_v, gsem, osem):
    wid = lax.axis_index("s") * NC + lax.axis_index("c")
    b0 = wid * B_PER_W

    pltpu.sync_copy(x_hbm.at[pl.ds(wid * NPAIR, NPAIR)], idx_v)

    def gather_start(p, rb):
        pltpu.async_copy(w_hbm.at[idx_v.at[p]], rows_v.at[rb], gsem)

    def gather_wait(p, rb):
        pltpu.make_async_copy(w_hbm.at[idx_v.at[p]], rows_v.at[rb], gsem).wait()

    def out_start(t, cb):
        pltpu.async_copy(comp_v.at[cb], out_hbm.at[pl.ds(b0 + GRP * t, GRP)], osem)

    def out_wait(t, cb):
        pltpu.make_async_copy(
            comp_v.at[cb], out_hbm.at[pl.ds(b0 + GRP * t, GRP)], osem
        ).wait()

    def scale_group(cb, rb_even, rb_odd):
        comp = comp_v.at[cb]
        for g in range(GRP):
            rows = rows_v.at[rb_even if g < 2 else rb_odd]
            base = HIST_PAD * (g % 2)

            @plsc.parallel_loop(0, HIST, unroll=4)
            def _(k):
                for i in range(D_MODEL // 16):
                    sl = pl.ds(16 * i, 16)
                    comp[g, k, sl] = rows[base + k, sl] * SCALE

    for rb in range(NRB):
        gather_start(rb, rb)

    def outer(tt, _):
        for u in range(NCB):
            t = tt * NCB + u
            p0 = 2 * t
            rb0 = (2 * u) % NRB
            rb1 = (2 * u + 1) % NRB
            cb = u
            gather_wait(p0, rb0)
            gather_wait(p0 + 1, rb1)

            @pl.when(t >= NCB)
            def _():
                out_wait(t - NCB, cb)

            scale_group(cb, rb0, rb1)
            out_start(t, cb)

            @pl.when(p0 + NRB + 1 < NPAIR + 1)
            def _():
                gather_start(p0 + NRB, rb0)
                gather_start(p0 + NRB + 1, rb1)

        return 0

    lax.fori_loop(0, NGRP // NCB, outer, 0)

    for u in range(NCB):
        out_wait(NGRP - NCB + u, u)


def kernel(x, weight):
    xp = jnp.concatenate([x, x[:, : HIST_PAD - HIST]], axis=1)
    xp = xp.reshape(BATCH // 2, PAIR)
    return _emb_sc(xp, weight)


# speedup vs baseline: 8.4221x; 1.0415x over previous
import functools
import math

import jax
import jax.numpy as jnp
from jax import lax
from jax.experimental import pallas as pl
from jax.experimental.pallas import tpu as pltpu
from jax.experimental.pallas import tpu_sc as plsc

VOCAB = 100000
D_MODEL = 128
BATCH = 4096
HIST = 50
PAIR = 2 * HIST
PAIR_PAD = 112

NC = 2
NS = 16
NW = NC * NS
B_PER_W = BATCH // NW
NPAIR = B_PER_W // 2
NRB = 8
SCALE = math.sqrt(D_MODEL)

_mesh = plsc.VectorSubcoreMesh(core_axis_name="c", subcore_axis_name="s")


@functools.partial(
    pl.kernel,
    mesh=_mesh,
    out_type=jax.ShapeDtypeStruct((BATCH, HIST, D_MODEL), jnp.float32),
    scratch_types=[
        pltpu.VMEM((NPAIR, PAIR_PAD), jnp.int32),
        pltpu.VMEM((NRB, PAIR, D_MODEL), jnp.float32),
        pltpu.SemaphoreType.DMA,
        pltpu.SemaphoreType.DMA,
    ],
)
def _emb_sc(x_hbm, w_hbm, out_hbm, idx_v, rows_v, gsem, osem):
    wid = lax.axis_index("s") * NC + lax.axis_index("c")
    b0 = wid * B_PER_W

    pltpu.sync_copy(x_hbm.at[pl.ds(wid * NPAIR, NPAIR)], idx_v)

    def gather_start(p, rb):
        pltpu.async_copy(
            w_hbm.at[idx_v.at[p, pl.ds(0, PAIR)]], rows_v.at[rb], gsem
        )

    def gather_wait(p, rb):
        pltpu.make_async_copy(
            w_hbm.at[idx_v.at[p, pl.ds(0, PAIR)]], rows_v.at[rb], gsem
        ).wait()

    def out_start(p, rb):
        pltpu.async_copy(
            rows_v.at[rb, pl.ds(0, HIST)], out_hbm.at[b0 + 2 * p], osem
        )
        pltpu.async_copy(
            rows_v.at[rb, pl.ds(HIST, HIST)], out_hbm.at[b0 + 2 * p + 1], osem
        )

    def out_wait(p, rb):
        pltpu.make_async_copy(
            rows_v.at[rb, pl.ds(0, HIST)], out_hbm.at[b0 + 2 * p], osem
        ).wait()
        pltpu.make_async_copy(
            rows_v.at[rb, pl.ds(HIST, HIST)], out_hbm.at[b0 + 2 * p + 1], osem
        ).wait()

    def scale_buf(rb):
        rows = rows_v.at[rb]

        @plsc.parallel_loop(0, PAIR, unroll=4)
        def _(k):
            for i in range(D_MODEL // 16):
                sl = pl.ds(16 * i, 16)
                rows[k, sl] = rows[k, sl] * SCALE

    for rb in range(NRB):
        gather_start(rb, rb)

    def outer(g, _):
        for rb in range(NRB):
            p = g * NRB + rb
            gather_wait(p, rb)
            scale_buf(rb)
            out_start(p, rb)
            nxt = p + NRB

            @pl.when(nxt < NPAIR)
            def _():
                out_wait(p, rb)
                gather_start(nxt, rb)

        return 0

    lax.fori_loop(0, NPAIR // NRB, outer, 0)

    for rb in range(NRB):
        out_wait(NPAIR - NRB + rb, rb)


def kernel(x, weight):
    xq = x.reshape(BATCH // 2, PAIR)
    xq = jnp.concatenate([xq, xq[:, : PAIR_PAD - PAIR]], axis=1)
    return _emb_sc(xq, weight)
